# baseline (speedup 1.0000x reference)
import functools

import jax
import jax.numpy as jnp
from jax import lax
from jax.experimental import pallas as pl
from jax.experimental.pallas import tpu as pltpu
from jax.experimental.pallas import tpu_sc as plsc

_NW = 32
_LANES = 16


@functools.lru_cache(maxsize=None)
def _build(batch, nf, vocab, emb):
    b_per_w = batch // _NW
    pairs_per_w = b_per_w * nf
    tab_field = vocab * emb
    n_bg = b_per_w // _LANES
    n_fpairs = nf // 2

    mesh = plsc.VectorSubcoreMesh(core_axis_name="c", subcore_axis_name="s")

    @functools.partial(
        pl.kernel,
        mesh=mesh,
        compiler_params=pltpu.CompilerParams(needs_layout_passes=False),
        out_type=jax.ShapeDtypeStruct((nf, emb, batch), jnp.float32),
        scratch_types=[
            pltpu.VMEM((pairs_per_w,), jnp.int32),
            pltpu.VMEM((tab_field,), jnp.float32),
            pltpu.VMEM((emb, b_per_w), jnp.float32),
            pltpu.VMEM((emb, b_per_w), jnp.float32),
            pltpu.VMEM_SHARED((nf * tab_field,), jnp.float32),
            pltpu.SemaphoreType.DMA,
            pltpu.SemaphoreType.DMA,
        ],
    )
    def k(xflat, tab1d, out, xc_v, tabf_v, obuf0, obuf1, tab_sh, s0, s1):
        cid = lax.axis_index("c")
        sid = lax.axis_index("s")
        wid = sid * 2 + cid
        b0 = wid * b_per_w

        @pl.when(sid == 0)
        def _():
            pltpu.sync_copy(tab1d, tab_sh)

        pltpu.sync_copy(xflat.at[pl.ds(wid * pairs_per_w, pairs_per_w)], xc_v)
        plsc.subcore_barrier()

        lanes_nf = lax.iota(jnp.int32, _LANES) * nf
        obufs = (obuf0, obuf1)
        sems = (s0, s1)

        def store(f, p):
            return pltpu.make_async_copy(
                obufs[p], out.at[f, :, pl.ds(b0, b_per_w)], sems[p]
            )

        def field_pair_body(fp, _):
            for p in range(2):
                f = fp * 2 + p
                obuf = obufs[p]

                pltpu.sync_copy(tab_sh.at[pl.ds(f * tab_field, tab_field)],
                                tabf_v)

                @pl.when(fp > 0)
                def _():
                    store(f - 2, p).wait()

                def bg_body(bg, _):
                    xidx = lanes_nf + (bg * (_LANES * nf) + f)
                    xv = plsc.load_gather(xc_v, [xidx])
                    xi = xv * emb
                    for d in range(emb):
                        obuf[d, pl.ds(bg * _LANES, _LANES)] = (
                            plsc.load_gather(tabf_v, [xi + d])
                        )
                    return 0

                lax.fori_loop(0, n_bg, bg_body, 0)
                store(f, p).start()
            return 0

        lax.fori_loop(0, n_fpairs, field_pair_body, 0)

        store(nf - 2, 0).wait()
        store(nf - 1, 1).wait()

    return k


def kernel(x_cat, tables):
    batch, nf = x_cat.shape
    _, vocab, emb = tables.shape
    xflat = x_cat.reshape(batch * nf).astype(jnp.int32)
    tab1d = tables.reshape(nf * vocab * emb)
    out = _build(batch, nf, vocab, emb)(xflat, tab1d)
    return jnp.transpose(out, (2, 0, 1))


# speedup vs baseline: 1.5358x; 1.5358x over previous
import functools

import jax
import jax.numpy as jnp
from jax import lax
from jax.experimental import pallas as pl
from jax.experimental.pallas import tpu as pltpu
from jax.experimental.pallas import tpu_sc as plsc

_NW = 32
_LANES = 16


@functools.lru_cache(maxsize=None)
def _build(batch, nf, vocab, emb):
    b_per_w = batch // _NW
    pairs_per_w = b_per_w * nf
    tab_field = vocab * emb
    n_bg = b_per_w // _LANES
    n_fpairs = nf // 2

    mesh = plsc.VectorSubcoreMesh(core_axis_name="c", subcore_axis_name="s")

    @functools.partial(
        pl.kernel,
        mesh=mesh,
        compiler_params=pltpu.CompilerParams(needs_layout_passes=False),
        out_type=jax.ShapeDtypeStruct((nf, emb, batch), jnp.float32),
        scratch_types=[
            pltpu.VMEM((pairs_per_w,), jnp.int32),
            pltpu.VMEM((tab_field,), jnp.float32),
            pltpu.VMEM((emb, b_per_w), jnp.float32),
            pltpu.VMEM((emb, b_per_w), jnp.float32),
            pltpu.VMEM_SHARED((nf * tab_field,), jnp.float32),
            pltpu.SemaphoreType.DMA,
            pltpu.SemaphoreType.DMA,
        ],
    )
    def k(xflat, tab1d, out, xc_v, tabf_v, obuf0, obuf1, tab_sh, s0, s1):
        cid = lax.axis_index("c")
        sid = lax.axis_index("s")
        wid = sid * 2 + cid
        b0 = wid * b_per_w

        @pl.when(sid == 0)
        def _():
            pltpu.sync_copy(tab1d, tab_sh)

        pltpu.sync_copy(xflat.at[pl.ds(wid * pairs_per_w, pairs_per_w)], xc_v)
        plsc.subcore_barrier()

        lanes_nf = lax.iota(jnp.int32, _LANES) * nf
        obufs = (obuf0, obuf1)
        sems = (s0, s1)

        def store(f, p):
            return pltpu.make_async_copy(
                obufs[p], out.at[f, :, pl.ds(b0, b_per_w)], sems[p]
            )

        def field_pair_body(fp, _):
            for p in range(2):
                f = fp * 2 + p
                obuf = obufs[p]

                pltpu.sync_copy(tab_sh.at[pl.ds(f * tab_field, tab_field)],
                                tabf_v)

                @pl.when(fp > 0)
                def _():
                    store(f - 2, p).wait()

                @plsc.parallel_loop(0, n_bg, unroll=2)
                def _(bg):
                    xidx = lanes_nf + (bg * (_LANES * nf) + f)
                    xv = plsc.load_gather(xc_v, [xidx])
                    xi = xv * emb
                    for d in range(emb):
                        obuf[d, pl.ds(bg * _LANES, _LANES)] = (
                            plsc.load_gather(tabf_v, [xi + d])
                        )
                store(f, p).start()
            return 0

        lax.fori_loop(0, n_fpairs, field_pair_body, 0)

        store(nf - 2, 0).wait()
        store(nf - 1, 1).wait()

    return k


def kernel(x_cat, tables):
    batch, nf = x_cat.shape
    _, vocab, emb = tables.shape
    xflat = x_cat.reshape(batch * nf).astype(jnp.int32)
    tab1d = tables.reshape(nf * vocab * emb)
    out = _build(batch, nf, vocab, emb)(xflat, tab1d)
    return jnp.transpose(out, (2, 0, 1))
